# baseline (speedup 1.0000x reference)
import functools

import jax
import jax.numpy as jnp
from jax import lax
from jax.experimental import pallas as pl
from jax.experimental.pallas import tpu as pltpu
from jax.experimental.pallas import tpu_sc as plsc

_NC = 2
_NS = 16
_NW = _NC * _NS
_D = 16
_L = 16
_R = 2.0
_T = 1.0

_NCOL = 7813
_CPT = 245
_NWAVE = 16
_WCOLS = 16
_HCAP = 1280
_WCAP = 128


def _pop(m):
    c = plsc.all_reduce_population_count(m)
    return c[0] if getattr(c, "shape", ()) else c


def _sc_scatter_rows(tT, idx_flat):
    n_idx = idx_flat.shape[0]
    v = tT.shape[1]
    n_steps = n_idx // _L
    out_rows = n_idx + _NW * _WCAP

    mesh = plsc.VectorSubcoreMesh(core_axis_name="c", subcore_axis_name="s")

    @functools.partial(
        pl.kernel,
        mesh=mesh,
        out_type=jax.ShapeDtypeStruct((out_rows * _D,), jnp.float32),
        compiler_params=pltpu.CompilerParams(needs_layout_passes=False),
        scratch_types=[
            pltpu.VMEM((n_idx,), jnp.int32),
            pltpu.VMEM((_HCAP,), jnp.int32),
            pltpu.VMEM((_HCAP,), jnp.int32),
            pltpu.VMEM((_WCAP,), jnp.int32),
            pltpu.VMEM((_WCAP,), jnp.int32),
            pltpu.VMEM((_WCOLS, _D, 128), jnp.float32),
            pltpu.VMEM((_WCOLS, _D, 128), jnp.float32),
            pltpu.VMEM((_WCAP * _D,), jnp.float32),
            pltpu.VMEM((_WCAP * _D,), jnp.float32),
            pltpu.VMEM((_WCAP * _D,), jnp.int32),
            pltpu.VMEM((_WCAP * _D,), jnp.int32),
            pltpu.SemaphoreType.DMA,
            pltpu.SemaphoreType.DMA,
            pltpu.SemaphoreType.DMA,
            pltpu.SemaphoreType.DMA,
        ],
    )
    def sc_kernel(tab_hbm, idx_hbm, out_hbm, idx_v, hv, hs, wv_val, wv_slot,
                  wb0, wb1, rs0, rs1, il0, il1, semw0, semw1, semr0, semr1):
        wid = lax.axis_index("s") * _NC + lax.axis_index("c")
        lo = wid * _CPT
        hi = jnp.minimum(lo + _CPT, _NCOL)

        pltpu.sync_copy(idx_hbm, idx_v)

        iota = lax.iota(jnp.int32, _L)

        def scan_body(st, cnt):
            vals = idx_v[pl.ds(st * _L, _L)]
            cols = vals >> 7
            m = (cols >= lo) & (cols < hi)
            plsc.store_compressed(hv.at[pl.ds(cnt, _L)], vals, mask=m)
            plsc.store_compressed(
                hs.at[pl.ds(cnt, _L)], st * _L + iota, mask=m
            )
            return cnt + _pop(m)

        cnt = lax.fori_loop(0, n_steps, scan_body, 0, unroll=2)

        def fire_wave(wv, wb, semw):
            c0 = lo + wv * _WCOLS
            for cc in range(_WCOLS):
                @pl.when(c0 + cc < hi)
                def _():
                    cbase = pl.multiple_of((c0 + cc) << 7, 128)
                    pltpu.async_copy(
                        tab_hbm.at[:, pl.ds(cbase, 128)], wb.at[cc], semw
                    )

        def drain_wave(wv, wb, semw):
            c0 = lo + wv * _WCOLS
            for cc in range(_WCOLS):
                @pl.when(c0 + cc < hi)
                def _():
                    cbase = pl.multiple_of((c0 + cc) << 7, 128)
                    pltpu.make_async_copy(
                        tab_hbm.at[:, pl.ds(cbase, 128)], wb.at[cc], semw
                    ).wait()

        def drain_rows(rstage, semr):
            pltpu.make_async_copy(
                rstage, out_hbm.at[pl.ds(0, _WCAP * _D)], semr
            ).wait()

        def process_wave(wv, wb, rstage, ilist, semr):
            c0 = lo + wv * _WCOLS
            lo_node = c0 << 7
            hi_node = jnp.minimum((c0 + _WCOLS) << 7, v) - 1

            def wbody(g, c):
                vals = hv[pl.ds(g * _L, _L)]
                slots = hs[pl.ds(g * _L, _L)]
                valid = (g * _L + iota) < cnt
                cr = (vals >> 7) - c0
                m = valid & (cr >= 0) & (cr < _WCOLS)
                plsc.store_compressed(wv_val.at[pl.ds(c, _L)], vals, mask=m)
                plsc.store_compressed(wv_slot.at[pl.ds(c, _L)], slots, mask=m)
                return c + _pop(m)

            cwv = lax.fori_loop(0, _HCAP // _L, wbody, 0, unroll=2)

            for g in range(_WCAP // _L):
                vvec = wv_val[pl.ds(g * _L, _L)]
                svec = wv_slot[pl.ds(g * _L, _L)]
                nodes = jnp.clip(vvec, lo_node, hi_node)
                cc_v = (nodes >> 7) - c0
                lane_v = nodes & 127
                valid = (g * _L + iota) < cwv
                dump = n_idx + wid * _WCAP + g * _L + iota
                slots = jnp.where(valid, svec, dump)
                gb = g * _L * _D
                for d in range(_D):
                    vals = plsc.load_gather(
                        wb, [cc_v, jnp.full((_L,), d, jnp.int32), lane_v]
                    )
                    rstage[pl.ds(gb + d * _L, _L)] = vals
                    ilist[pl.ds(gb + d * _L, _L)] = slots * _D + d
                for half in range(2):
                    off = gb + half * 128
                    pltpu.async_copy(
                        rstage.at[pl.ds(off, 128)],
                        out_hbm.at[ilist.at[pl.ds(off, 128)]],
                        semr,
                    )

        fire_wave(0, wb0, semw0)
        fire_wave(1, wb1, semw1)

        @pl.loop(0, _NWAVE, step=2)
        def _(j):
            drain_wave(j, wb0, semw0)

            @pl.when(j >= 2)
            def _():
                drain_rows(rs0, semr0)

            process_wave(j, wb0, rs0, il0, semr0)

            @pl.when(j + 2 < _NWAVE)
            def _():
                fire_wave(j + 2, wb0, semw0)

            drain_wave(j + 1, wb1, semw1)

            @pl.when(j >= 2)
            def _():
                drain_rows(rs1, semr1)

            process_wave(j + 1, wb1, rs1, il1, semr1)

            @pl.when(j + 3 < _NWAVE)
            def _():
                fire_wave(j + 3, wb1, semw1)

        drain_rows(rs0, semr0)
        drain_rows(rs1, semr1)

    return sc_kernel(tT, idx_flat)


def _sc_pair_x(rows, n_pairs):
    per_w = (2 * n_pairs) // _NW
    kc = per_w // 128
    pairs_per_w = per_w // 2

    mesh = plsc.VectorSubcoreMesh(core_axis_name="c", subcore_axis_name="s")

    @functools.partial(
        pl.kernel,
        mesh=mesh,
        out_type=jax.ShapeDtypeStruct((n_pairs,), jnp.float32),
        compiler_params=pltpu.CompilerParams(needs_layout_passes=False),
        scratch_types=[
            pltpu.VMEM((128 * _D,), jnp.float32),
            pltpu.VMEM((128 * _D,), jnp.float32),
            pltpu.VMEM((pairs_per_w,), jnp.float32),
            pltpu.SemaphoreType.DMA,
            pltpu.SemaphoreType.DMA,
        ],
    )
    def sc_kernel(rows_hbm, out_hbm, buf0, buf1, x_v, sem0, sem1):
        wid = lax.axis_index("s") * _NC + lax.axis_index("c")
        base = wid * per_w * _D

        iota = lax.iota(jnp.int32, _L)
        u_base = iota * (2 * _D)
        v_base = u_base + _D
        zero = jnp.zeros((_L,), jnp.float32)

        def fire(jj, buf, sem):
            pltpu.async_copy(
                rows_hbm.at[pl.ds(base + jj * 128 * _D, 128 * _D)], buf, sem
            )

        def compute(jj, buf, sem):
            pltpu.make_async_copy(
                rows_hbm.at[pl.ds(base + jj * 128 * _D, 128 * _D)], buf, sem
            ).wait()
            for blk in range(4):
                cb = blk * 2 * _L * _D
                e2 = zero
                un = zero
                vn = zero
                for d in range(_D):
                    uc = plsc.load_gather(buf, [cb + u_base + d])
                    vc = plsc.load_gather(buf, [cb + v_base + d])
                    df = uc - vc
                    e2 = e2 + df * df
                    un = un + uc * uc
                    vn = vn + vc * vc
                xblk = 1.0 + 2.0 * e2 / ((1.0 - un) * (1.0 - vn))
                x_v[pl.ds((jj * 4 + blk) * _L, _L)] = xblk

        fire(0, buf0, sem0)
        fire(1, buf1, sem1)

        @pl.loop(0, kc, step=2)
        def _(j):
            compute(j, buf0, sem0)

            @pl.when(j + 2 < kc)
            def _():
                fire(j + 2, buf0, sem0)

            compute(j + 1, buf1, sem1)

            @pl.when(j + 3 < kc)
            def _():
                fire(j + 3, buf1, sem1)

        pltpu.sync_copy(x_v, out_hbm.at[pl.ds(wid * pairs_per_w, pairs_per_w)])

    return sc_kernel(rows)


def _tc_tail_body(x_ref, lab_ref, out_ref):
    x = x_ref[...]
    dist = jnp.log(x + jnp.sqrt(x * x - 1.0))
    z = jnp.exp((dist - _R) / _T)
    lab = lab_ref[...]
    out_ref[...] = jnp.where(lab == 1, jnp.log(z + 1.0), jnp.log(1.0 + 1.0 / z))


def _tc_tail(x, labels):
    b = labels.shape[0]
    r = b // 128
    out = pl.pallas_call(
        _tc_tail_body,
        out_shape=jax.ShapeDtypeStruct((r, 128), jnp.float32),
    )(x.reshape(r, 128), labels.reshape(r, 128))
    return out.reshape(b)


@jax.jit
def kernel(pairs, labels, table):
    b = pairs.shape[0]
    tT = table.T
    idx = pairs.reshape(-1)
    rows = _sc_scatter_rows(tT, idx)
    x = _sc_pair_x(rows, b)
    return _tc_tail(x, labels)


# speedup vs baseline: 17.0182x; 17.0182x over previous
import functools

import jax
import jax.numpy as jnp
from jax import lax
from jax.experimental import pallas as pl
from jax.experimental.pallas import tpu as pltpu
from jax.experimental.pallas import tpu_sc as plsc

_NC = 2
_NS = 16
_NW = _NC * _NS
_D = 16
_L = 16
_R = 2.0
_T = 1.0

_NCOL = 7813
_CPT = 245
_NWAVE = 16
_WCOLS = 16
_HCAP = 1280
_WCAP = 128


def _pop(m):
    c = plsc.all_reduce_population_count(m)
    return c[0] if getattr(c, "shape", ()) else c


def _sc_scatter_rows(tT, idx_flat):
    n_idx = idx_flat.shape[0]
    v = tT.shape[1]
    n_steps = n_idx // _L
    out_rows = n_idx + _NW * _WCAP

    mesh = plsc.VectorSubcoreMesh(core_axis_name="c", subcore_axis_name="s")

    @functools.partial(
        pl.kernel,
        mesh=mesh,
        out_type=jax.ShapeDtypeStruct((out_rows * _D,), jnp.float32),
        compiler_params=pltpu.CompilerParams(needs_layout_passes=False),
        scratch_types=[
            pltpu.VMEM((n_idx,), jnp.int32),
            pltpu.VMEM((_HCAP,), jnp.int32),
            pltpu.VMEM((_HCAP,), jnp.int32),
            pltpu.VMEM((_NWAVE * _WCAP,), jnp.int32),
            pltpu.VMEM((_NWAVE * _WCAP,), jnp.int32),
            pltpu.VMEM((_WCOLS, _D, 128), jnp.float32),
            pltpu.VMEM((_WCOLS, _D, 128), jnp.float32),
            pltpu.VMEM((_WCAP * _D,), jnp.float32),
            pltpu.VMEM((_WCAP * _D,), jnp.float32),
            pltpu.SemaphoreType.DMA,
            pltpu.SemaphoreType.DMA,
            pltpu.SemaphoreType.DMA,
            pltpu.SemaphoreType.DMA,
        ],
    )
    def sc_kernel(tab_hbm, idx_hbm, out_hbm, idx_v, hv, hs, bv_f, bs_f,
                  wb0, wb1, rs0, rs1, semw0, semw1, semr0, semr1):
        wid = lax.axis_index("s") * _NC + lax.axis_index("c")
        lo = wid * _CPT
        hi = jnp.minimum(lo + _CPT, _NCOL)

        pltpu.sync_copy(idx_hbm, idx_v)

        iota = lax.iota(jnp.int32, _L)

        def scan_body(st, cnt):
            vals = idx_v[pl.ds(st * _L, _L)]
            cols = vals >> 7
            m = (cols >= lo) & (cols < hi)
            plsc.store_compressed(hv.at[pl.ds(cnt, _L)], vals, mask=m)
            plsc.store_compressed(
                hs.at[pl.ds(cnt, _L)], st * _L + iota, mask=m
            )
            return cnt + _pop(m)

        cnt = lax.fori_loop(0, n_steps, scan_body, 0, unroll=2)

        for g in range(_WCAP // _L):
            dv = n_idx + wid * _WCAP + g * _L + iota
            zv = jnp.zeros((_L,), jnp.int32)
            for w in range(_NWAVE):
                bs_f[pl.ds(w * _WCAP + g * _L, _L)] = dv
                bv_f[pl.ds(w * _WCAP + g * _L, _L)] = zv

        def bucket_body(g, cws):
            vals = hv[pl.ds(g * _L, _L)]
            slots = hs[pl.ds(g * _L, _L)]
            valid = (g * _L + iota) < cnt
            wvid = jnp.clip(((vals >> 7) - lo) >> 4, 0, _NWAVE - 1)
            out = []
            for w in range(_NWAVE):
                m = valid & (wvid == w)
                plsc.store_compressed(
                    bv_f.at[pl.ds(w * _WCAP + cws[w], _L)], vals, mask=m
                )
                plsc.store_compressed(
                    bs_f.at[pl.ds(w * _WCAP + cws[w], _L)], slots, mask=m
                )
                out.append(jnp.minimum(cws[w] + _pop(m), _WCAP - _L))
            return tuple(out)

        lax.fori_loop(0, _HCAP // _L, bucket_body, (0,) * _NWAVE)

        def fire_wave(wv, wb, semw):
            c0 = lo + wv * _WCOLS
            for cc in range(_WCOLS):
                @pl.when(c0 + cc < hi)
                def _():
                    cbase = pl.multiple_of((c0 + cc) << 7, 128)
                    pltpu.async_copy(
                        tab_hbm.at[:, pl.ds(cbase, 128)], wb.at[cc], semw
                    )

        def drain_wave(wv, wb, semw):
            c0 = lo + wv * _WCOLS
            for cc in range(_WCOLS):
                @pl.when(c0 + cc < hi)
                def _():
                    cbase = pl.multiple_of((c0 + cc) << 7, 128)
                    pltpu.make_async_copy(
                        tab_hbm.at[:, pl.ds(cbase, 128)], wb.at[cc], semw
                    ).wait()

        def drain_rows(rstage, semr):
            pltpu.make_async_copy(
                rstage, out_hbm.at[pl.ds(0, _WCAP * _D)], semr
            ).wait()

        def process_wave(wv, wb, rstage, semr):
            c0 = lo + wv * _WCOLS
            lo_node = c0 << 7
            hi_node = jnp.minimum((c0 + _WCOLS) << 7, v) - 1

            for g in range(_WCAP // _L):
                vvec = bv_f[pl.ds(wv * _WCAP + g * _L, _L)]
                slots = bs_f[pl.ds(wv * _WCAP + g * _L, _L)]
                nodes = jnp.clip(vvec, lo_node, hi_node)
                cc_v = (nodes >> 7) - c0
                lane_v = nodes & 127
                gb = g * _L * _D
                for d in range(_D):
                    vals = plsc.load_gather(
                        wb, [cc_v, jnp.full((_L,), d, jnp.int32), lane_v]
                    )
                    plsc.store_scatter(rstage, [gb + iota * _D + d], vals)
                for t in range(_L):
                    slot = slots[t]
                    pltpu.async_copy(
                        rstage.at[pl.ds(gb + t * _D, _D)],
                        out_hbm.at[pl.ds(slot * _D, _D)],
                        semr,
                    )

        fire_wave(0, wb0, semw0)
        fire_wave(1, wb1, semw1)

        @pl.loop(0, _NWAVE, step=2)
        def _(j):
            drain_wave(j, wb0, semw0)

            @pl.when(j >= 2)
            def _():
                drain_rows(rs0, semr0)

            process_wave(j, wb0, rs0, semr0)

            @pl.when(j + 2 < _NWAVE)
            def _():
                fire_wave(j + 2, wb0, semw0)

            drain_wave(j + 1, wb1, semw1)

            @pl.when(j >= 2)
            def _():
                drain_rows(rs1, semr1)

            process_wave(j + 1, wb1, rs1, semr1)

            @pl.when(j + 3 < _NWAVE)
            def _():
                fire_wave(j + 3, wb1, semw1)

        drain_rows(rs0, semr0)
        drain_rows(rs1, semr1)

    return sc_kernel(tT, idx_flat)


def _sc_pair_x(rows, n_pairs):
    per_w = (2 * n_pairs) // _NW
    kc = per_w // 128
    pairs_per_w = per_w // 2

    mesh = plsc.VectorSubcoreMesh(core_axis_name="c", subcore_axis_name="s")

    @functools.partial(
        pl.kernel,
        mesh=mesh,
        out_type=jax.ShapeDtypeStruct((n_pairs,), jnp.float32),
        compiler_params=pltpu.CompilerParams(needs_layout_passes=False),
        scratch_types=[
            pltpu.VMEM((128 * _D,), jnp.float32),
            pltpu.VMEM((128 * _D,), jnp.float32),
            pltpu.VMEM((pairs_per_w,), jnp.float32),
            pltpu.SemaphoreType.DMA,
            pltpu.SemaphoreType.DMA,
        ],
    )
    def sc_kernel(rows_hbm, out_hbm, buf0, buf1, x_v, sem0, sem1):
        wid = lax.axis_index("s") * _NC + lax.axis_index("c")
        base = wid * per_w * _D

        iota = lax.iota(jnp.int32, _L)
        u_base = iota * (2 * _D)
        v_base = u_base + _D
        zero = jnp.zeros((_L,), jnp.float32)

        def fire(jj, buf, sem):
            pltpu.async_copy(
                rows_hbm.at[pl.ds(base + jj * 128 * _D, 128 * _D)], buf, sem
            )

        def compute(jj, buf, sem):
            pltpu.make_async_copy(
                rows_hbm.at[pl.ds(base + jj * 128 * _D, 128 * _D)], buf, sem
            ).wait()
            for blk in range(4):
                cb = blk * 2 * _L * _D
                e2 = zero
                un = zero
                vn = zero
                for d in range(_D):
                    uc = plsc.load_gather(buf, [cb + u_base + d])
                    vc = plsc.load_gather(buf, [cb + v_base + d])
                    df = uc - vc
                    e2 = e2 + df * df
                    un = un + uc * uc
                    vn = vn + vc * vc
                xblk = 1.0 + 2.0 * e2 / ((1.0 - un) * (1.0 - vn))
                x_v[pl.ds((jj * 4 + blk) * _L, _L)] = xblk

        fire(0, buf0, sem0)
        fire(1, buf1, sem1)

        @pl.loop(0, kc, step=2)
        def _(j):
            compute(j, buf0, sem0)

            @pl.when(j + 2 < kc)
            def _():
                fire(j + 2, buf0, sem0)

            compute(j + 1, buf1, sem1)

            @pl.when(j + 3 < kc)
            def _():
                fire(j + 3, buf1, sem1)

        pltpu.sync_copy(x_v, out_hbm.at[pl.ds(wid * pairs_per_w, pairs_per_w)])

    return sc_kernel(rows)


def _tc_tail_body(x_ref, lab_ref, out_ref):
    x = x_ref[...]
    dist = jnp.log(x + jnp.sqrt(x * x - 1.0))
    z = jnp.exp((dist - _R) / _T)
    lab = lab_ref[...]
    out_ref[...] = jnp.where(lab == 1, jnp.log(z + 1.0), jnp.log(1.0 + 1.0 / z))


def _tc_tail(x, labels):
    b = labels.shape[0]
    r = b // 128
    out = pl.pallas_call(
        _tc_tail_body,
        out_shape=jax.ShapeDtypeStruct((r, 128), jnp.float32),
    )(x.reshape(r, 128), labels.reshape(r, 128))
    return out.reshape(b)


@jax.jit
def kernel(pairs, labels, table):
    b = pairs.shape[0]
    tT = table.T
    idx = pairs.reshape(-1)
    rows = _sc_scatter_rows(tT, idx)
    x = _sc_pair_x(rows, b)
    return _tc_tail(x, labels)


# speedup vs baseline: 17.9481x; 1.0546x over previous
import functools

import jax
import jax.numpy as jnp
from jax import lax
from jax.experimental import pallas as pl
from jax.experimental.pallas import tpu as pltpu
from jax.experimental.pallas import tpu_sc as plsc

_NC = 2
_NS = 16
_NW = _NC * _NS
_D = 16
_L = 16
_R = 2.0
_T = 1.0

_NCOL = 7813
_CPT = 245
_NWAVE = 16
_WCOLS = 16
_HCAP = 1280
_WCAP = 112


def _pop(m):
    c = plsc.all_reduce_population_count(m)
    return c[0] if getattr(c, "shape", ()) else c


def _sc_scatter_rows(tT, idx_flat):
    n_idx = idx_flat.shape[0]
    v = tT.shape[1]
    n_steps = n_idx // _L
    out_rows = n_idx + _NW * _WCAP

    mesh = plsc.VectorSubcoreMesh(core_axis_name="c", subcore_axis_name="s")

    @functools.partial(
        pl.kernel,
        mesh=mesh,
        out_type=jax.ShapeDtypeStruct((out_rows * _D,), jnp.float32),
        compiler_params=pltpu.CompilerParams(needs_layout_passes=False),
        scratch_types=[
            pltpu.VMEM((n_idx,), jnp.int32),
            pltpu.VMEM((_HCAP,), jnp.int32),
            pltpu.VMEM((_HCAP,), jnp.int32),
            pltpu.VMEM((_NWAVE * _WCAP,), jnp.int32),
            pltpu.VMEM((_NWAVE * _WCAP,), jnp.int32),
            pltpu.VMEM((_WCOLS, _D, 128), jnp.float32),
            pltpu.VMEM((_WCOLS, _D, 128), jnp.float32),
            pltpu.VMEM((_WCAP * _D,), jnp.float32),
            pltpu.VMEM((_WCAP * _D,), jnp.float32),
            pltpu.SemaphoreType.DMA,
            pltpu.SemaphoreType.DMA,
            pltpu.SemaphoreType.DMA,
            pltpu.SemaphoreType.DMA,
        ],
    )
    def sc_kernel(tab_hbm, idx_hbm, out_hbm, idx_v, hv, hs, bv_f, bs_f,
                  wb0, wb1, rs0, rs1, semw0, semw1, semr0, semr1):
        wid = lax.axis_index("s") * _NC + lax.axis_index("c")
        lo = wid * _CPT
        hi = jnp.minimum(lo + _CPT, _NCOL)

        pltpu.sync_copy(idx_hbm, idx_v)

        iota = lax.iota(jnp.int32, _L)

        def fire_wave(wv, wb, semw):
            c0 = lo + wv * _WCOLS
            for cc in range(_WCOLS):
                @pl.when(c0 + cc < hi)
                def _():
                    cbase = pl.multiple_of((c0 + cc) << 7, 128)
                    pltpu.async_copy(
                        tab_hbm.at[:, pl.ds(cbase, 128)], wb.at[cc], semw
                    )

        fire_wave(0, wb0, semw0)
        fire_wave(1, wb1, semw1)

        def scan_body(st, cnt):
            vals = idx_v[pl.ds(st * _L, _L)]
            cols = vals >> 7
            m = (cols >= lo) & (cols < hi)
            plsc.store_compressed(hv.at[pl.ds(cnt, _L)], vals, mask=m)
            plsc.store_compressed(
                hs.at[pl.ds(cnt, _L)], st * _L + iota, mask=m
            )
            return cnt + _pop(m)

        cnt = lax.fori_loop(0, n_steps, scan_body, 0, unroll=2)

        for g in range(_WCAP // _L):
            dv = n_idx + wid * _WCAP + g * _L + iota
            zv = jnp.zeros((_L,), jnp.int32)
            for w in range(_NWAVE):
                bs_f[pl.ds(w * _WCAP + g * _L, _L)] = dv
                bv_f[pl.ds(w * _WCAP + g * _L, _L)] = zv

        def bucket_body(g, cws):
            vals = hv[pl.ds(g * _L, _L)]
            slots = hs[pl.ds(g * _L, _L)]
            valid = (g * _L + iota) < cnt
            wvid = jnp.clip(((vals >> 7) - lo) >> 4, 0, _NWAVE - 1)
            out = []
            for w in range(_NWAVE):
                m = valid & (wvid == w)
                plsc.store_compressed(
                    bv_f.at[pl.ds(w * _WCAP + cws[w], _L)], vals, mask=m
                )
                plsc.store_compressed(
                    bs_f.at[pl.ds(w * _WCAP + cws[w], _L)], slots, mask=m
                )
                out.append(jnp.minimum(cws[w] + _pop(m), _WCAP - _L))
            return tuple(out)

        lax.fori_loop(0, _HCAP // _L, bucket_body, (0,) * _NWAVE)

        def drain_wave(wv, wb, semw):
            c0 = lo + wv * _WCOLS
            for cc in range(_WCOLS):
                @pl.when(c0 + cc < hi)
                def _():
                    cbase = pl.multiple_of((c0 + cc) << 7, 128)
                    pltpu.make_async_copy(
                        tab_hbm.at[:, pl.ds(cbase, 128)], wb.at[cc], semw
                    ).wait()

        def drain_rows(rstage, semr):
            pltpu.make_async_copy(
                rstage, out_hbm.at[pl.ds(0, _WCAP * _D)], semr
            ).wait()

        def process_wave(wv, wb, rstage, semr):
            c0 = lo + wv * _WCOLS
            lo_node = c0 << 7
            hi_node = jnp.minimum((c0 + _WCOLS) << 7, v) - 1

            for g in range(_WCAP // _L):
                vvec = bv_f[pl.ds(wv * _WCAP + g * _L, _L)]
                slots = bs_f[pl.ds(wv * _WCAP + g * _L, _L)]
                nodes = jnp.clip(vvec, lo_node, hi_node)
                cc_v = (nodes >> 7) - c0
                lane_v = nodes & 127
                gb = g * _L * _D
                for d in range(_D):
                    vals = plsc.load_gather(
                        wb, [cc_v, jnp.full((_L,), d, jnp.int32), lane_v]
                    )
                    plsc.store_scatter(rstage, [gb + iota * _D + d], vals)
                for t in range(_L):
                    slot = slots[t]
                    pltpu.async_copy(
                        rstage.at[pl.ds(gb + t * _D, _D)],
                        out_hbm.at[pl.ds(slot * _D, _D)],
                        semr,
                    )

        @pl.loop(0, _NWAVE, step=2)
        def _(j):
            drain_wave(j, wb0, semw0)

            @pl.when(j >= 2)
            def _():
                drain_rows(rs0, semr0)

            process_wave(j, wb0, rs0, semr0)

            @pl.when(j + 2 < _NWAVE)
            def _():
                fire_wave(j + 2, wb0, semw0)

            drain_wave(j + 1, wb1, semw1)

            @pl.when(j >= 2)
            def _():
                drain_rows(rs1, semr1)

            process_wave(j + 1, wb1, rs1, semr1)

            @pl.when(j + 3 < _NWAVE)
            def _():
                fire_wave(j + 3, wb1, semw1)

        drain_rows(rs0, semr0)
        drain_rows(rs1, semr1)

    return sc_kernel(tT, idx_flat)


def _sc_pair_x(rows, n_pairs):
    per_w = (2 * n_pairs) // _NW
    kc = per_w // 128
    pairs_per_w = per_w // 2

    mesh = plsc.VectorSubcoreMesh(core_axis_name="c", subcore_axis_name="s")

    @functools.partial(
        pl.kernel,
        mesh=mesh,
        out_type=jax.ShapeDtypeStruct((n_pairs,), jnp.float32),
        compiler_params=pltpu.CompilerParams(needs_layout_passes=False),
        scratch_types=[
            pltpu.VMEM((128 * _D,), jnp.float32),
            pltpu.VMEM((128 * _D,), jnp.float32),
            pltpu.VMEM((pairs_per_w,), jnp.float32),
            pltpu.SemaphoreType.DMA,
            pltpu.SemaphoreType.DMA,
        ],
    )
    def sc_kernel(rows_hbm, out_hbm, buf0, buf1, x_v, sem0, sem1):
        wid = lax.axis_index("s") * _NC + lax.axis_index("c")
        base = wid * per_w * _D

        iota = lax.iota(jnp.int32, _L)
        u_base = iota * (2 * _D)
        v_base = u_base + _D
        zero = jnp.zeros((_L,), jnp.float32)

        def fire(jj, buf, sem):
            pltpu.async_copy(
                rows_hbm.at[pl.ds(base + jj * 128 * _D, 128 * _D)], buf, sem
            )

        def compute(jj, buf, sem):
            pltpu.make_async_copy(
                rows_hbm.at[pl.ds(base + jj * 128 * _D, 128 * _D)], buf, sem
            ).wait()
            for blk in range(4):
                cb = blk * 2 * _L * _D
                e2 = zero
                un = zero
                vn = zero
                for d in range(_D):
                    uc = plsc.load_gather(buf, [cb + u_base + d])
                    vc = plsc.load_gather(buf, [cb + v_base + d])
                    df = uc - vc
                    e2 = e2 + df * df
                    un = un + uc * uc
                    vn = vn + vc * vc
                xblk = 1.0 + 2.0 * e2 / ((1.0 - un) * (1.0 - vn))
                x_v[pl.ds((jj * 4 + blk) * _L, _L)] = xblk

        fire(0, buf0, sem0)
        fire(1, buf1, sem1)

        @pl.loop(0, kc, step=2)
        def _(j):
            compute(j, buf0, sem0)

            @pl.when(j + 2 < kc)
            def _():
                fire(j + 2, buf0, sem0)

            compute(j + 1, buf1, sem1)

            @pl.when(j + 3 < kc)
            def _():
                fire(j + 3, buf1, sem1)

        pltpu.sync_copy(x_v, out_hbm.at[pl.ds(wid * pairs_per_w, pairs_per_w)])

    return sc_kernel(rows)


def _tc_tail_body(x_ref, lab_ref, out_ref):
    x = x_ref[...]
    dist = jnp.log(x + jnp.sqrt(x * x - 1.0))
    z = jnp.exp((dist - _R) / _T)
    lab = lab_ref[...]
    out_ref[...] = jnp.where(lab == 1, jnp.log(z + 1.0), jnp.log(1.0 + 1.0 / z))


def _tc_tail(x, labels):
    b = labels.shape[0]
    r = b // 128
    out = pl.pallas_call(
        _tc_tail_body,
        out_shape=jax.ShapeDtypeStruct((r, 128), jnp.float32),
    )(x.reshape(r, 128), labels.reshape(r, 128))
    return out.reshape(b)


@jax.jit
def kernel(pairs, labels, table):
    b = pairs.shape[0]
    tT = table.T
    idx = pairs.reshape(-1)
    rows = _sc_scatter_rows(tT, idx)
    x = _sc_pair_x(rows, b)
    return _tc_tail(x, labels)


# speedup vs baseline: 18.0960x; 1.0082x over previous
import functools

import jax
import jax.numpy as jnp
from jax import lax
from jax.experimental import pallas as pl
from jax.experimental.pallas import tpu as pltpu
from jax.experimental.pallas import tpu_sc as plsc

_NC = 2
_NS = 16
_NW = _NC * _NS
_D = 16
_L = 16
_R = 2.0
_T = 1.0

_NCOL = 7813
_CPT = 245
_NWAVE = 16
_WCOLS = 16
_HCAP = 1280
_WCAP = 112


def _pop(m):
    c = plsc.all_reduce_population_count(m)
    return c[0] if getattr(c, "shape", ()) else c


def _sc_scatter_rows(tT, idx_flat):
    n_idx = idx_flat.shape[0]
    v = tT.shape[1]
    n_steps = n_idx // _L
    out_rows = n_idx + _NW * _WCAP

    mesh = plsc.VectorSubcoreMesh(core_axis_name="c", subcore_axis_name="s")

    @functools.partial(
        pl.kernel,
        mesh=mesh,
        out_type=jax.ShapeDtypeStruct((out_rows * _D,), jnp.float32),
        compiler_params=pltpu.CompilerParams(needs_layout_passes=False),
        scratch_types=[
            pltpu.VMEM((n_idx,), jnp.int32),
            pltpu.VMEM((_HCAP,), jnp.int32),
            pltpu.VMEM((_HCAP,), jnp.int32),
            pltpu.VMEM((_NWAVE * _WCAP,), jnp.int32),
            pltpu.VMEM((_NWAVE * _WCAP,), jnp.int32),
            pltpu.VMEM((_WCOLS, _D, 128), jnp.float32),
            pltpu.VMEM((_WCOLS, _D, 128), jnp.float32),
            pltpu.VMEM((_WCAP * _D,), jnp.float32),
            pltpu.VMEM((_WCAP * _D,), jnp.float32),
            pltpu.SemaphoreType.DMA,
            pltpu.SemaphoreType.DMA,
            pltpu.SemaphoreType.DMA,
            pltpu.SemaphoreType.DMA,
        ],
    )
    def sc_kernel(tab_hbm, idx_hbm, out_hbm, idx_v, hv, hs, bv_f, bs_f,
                  wb0, wb1, rs0, rs1, semw0, semw1, semr0, semr1):
        wid = lax.axis_index("s") * _NC + lax.axis_index("c")
        lo = wid * _CPT
        hi = jnp.minimum(lo + _CPT, _NCOL)

        pltpu.sync_copy(idx_hbm, idx_v)

        iota = lax.iota(jnp.int32, _L)

        def fire_wave(wv, wb, semw):
            c0 = lo + wv * _WCOLS
            for cc in range(_WCOLS):
                @pl.when(c0 + cc < hi)
                def _():
                    cbase = pl.multiple_of((c0 + cc) << 7, 128)
                    pltpu.async_copy(
                        tab_hbm.at[:, pl.ds(cbase, 128)], wb.at[cc], semw
                    )

        fire_wave(0, wb0, semw0)
        fire_wave(1, wb1, semw1)

        def scan_body(st, cnt):
            vals = idx_v[pl.ds(st * _L, _L)]
            cols = vals >> 7
            m = (cols >= lo) & (cols < hi)
            plsc.store_compressed(hv.at[pl.ds(cnt, _L)], vals, mask=m)
            plsc.store_compressed(
                hs.at[pl.ds(cnt, _L)], st * _L + iota, mask=m
            )
            return cnt + _pop(m)

        cnt = lax.fori_loop(0, n_steps, scan_body, 0, unroll=4)

        for g in range(_WCAP // _L):
            dv = n_idx + wid * _WCAP + g * _L + iota
            zv = jnp.zeros((_L,), jnp.int32)
            for w in range(_NWAVE):
                bs_f[pl.ds(w * _WCAP + g * _L, _L)] = dv
                bv_f[pl.ds(w * _WCAP + g * _L, _L)] = zv

        def bucket_body(g, cws):
            vals = hv[pl.ds(g * _L, _L)]
            slots = hs[pl.ds(g * _L, _L)]
            valid = (g * _L + iota) < cnt
            wvid = jnp.clip(((vals >> 7) - lo) >> 4, 0, _NWAVE - 1)
            out = []
            for w in range(_NWAVE):
                m = valid & (wvid == w)
                plsc.store_compressed(
                    bv_f.at[pl.ds(w * _WCAP + cws[w], _L)], vals, mask=m
                )
                plsc.store_compressed(
                    bs_f.at[pl.ds(w * _WCAP + cws[w], _L)], slots, mask=m
                )
                out.append(jnp.minimum(cws[w] + _pop(m), _WCAP - _L))
            return tuple(out)

        lax.fori_loop(0, (cnt + _L - 1) // _L, bucket_body, (0,) * _NWAVE)

        def drain_wave(wv, wb, semw):
            c0 = lo + wv * _WCOLS
            for cc in range(_WCOLS):
                @pl.when(c0 + cc < hi)
                def _():
                    cbase = pl.multiple_of((c0 + cc) << 7, 128)
                    pltpu.make_async_copy(
                        tab_hbm.at[:, pl.ds(cbase, 128)], wb.at[cc], semw
                    ).wait()

        def drain_rows(rstage, semr):
            pltpu.make_async_copy(
                rstage, out_hbm.at[pl.ds(0, _WCAP * _D)], semr
            ).wait()

        def process_wave(wv, wb, rstage, semr):
            c0 = lo + wv * _WCOLS
            lo_node = c0 << 7
            hi_node = jnp.minimum((c0 + _WCOLS) << 7, v) - 1

            for g in range(_WCAP // _L):
                vvec = bv_f[pl.ds(wv * _WCAP + g * _L, _L)]
                slots = bs_f[pl.ds(wv * _WCAP + g * _L, _L)]
                nodes = jnp.clip(vvec, lo_node, hi_node)
                cc_v = (nodes >> 7) - c0
                lane_v = nodes & 127
                gb = g * _L * _D
                for d in range(_D):
                    vals = plsc.load_gather(
                        wb, [cc_v, jnp.full((_L,), d, jnp.int32), lane_v]
                    )
                    plsc.store_scatter(rstage, [gb + iota * _D + d], vals)
                for t in range(_L):
                    slot = slots[t]
                    pltpu.async_copy(
                        rstage.at[pl.ds(gb + t * _D, _D)],
                        out_hbm.at[pl.ds(slot * _D, _D)],
                        semr,
                    )

        @pl.loop(0, _NWAVE, step=2)
        def _(j):
            drain_wave(j, wb0, semw0)

            @pl.when(j >= 2)
            def _():
                drain_rows(rs0, semr0)

            process_wave(j, wb0, rs0, semr0)

            @pl.when(j + 2 < _NWAVE)
            def _():
                fire_wave(j + 2, wb0, semw0)

            drain_wave(j + 1, wb1, semw1)

            @pl.when(j >= 2)
            def _():
                drain_rows(rs1, semr1)

            process_wave(j + 1, wb1, rs1, semr1)

            @pl.when(j + 3 < _NWAVE)
            def _():
                fire_wave(j + 3, wb1, semw1)

        drain_rows(rs0, semr0)
        drain_rows(rs1, semr1)

    return sc_kernel(tT, idx_flat)


def _sc_pair_x(rows, n_pairs):
    per_w = (2 * n_pairs) // _NW
    kc = per_w // 128
    pairs_per_w = per_w // 2

    mesh = plsc.VectorSubcoreMesh(core_axis_name="c", subcore_axis_name="s")

    @functools.partial(
        pl.kernel,
        mesh=mesh,
        out_type=jax.ShapeDtypeStruct((n_pairs,), jnp.float32),
        compiler_params=pltpu.CompilerParams(needs_layout_passes=False),
        scratch_types=[
            pltpu.VMEM((128 * _D,), jnp.float32),
            pltpu.VMEM((128 * _D,), jnp.float32),
            pltpu.VMEM((pairs_per_w,), jnp.float32),
            pltpu.SemaphoreType.DMA,
            pltpu.SemaphoreType.DMA,
        ],
    )
    def sc_kernel(rows_hbm, out_hbm, buf0, buf1, x_v, sem0, sem1):
        wid = lax.axis_index("s") * _NC + lax.axis_index("c")
        base = wid * per_w * _D

        iota = lax.iota(jnp.int32, _L)
        u_base = iota * (2 * _D)
        v_base = u_base + _D
        zero = jnp.zeros((_L,), jnp.float32)

        def fire(jj, buf, sem):
            pltpu.async_copy(
                rows_hbm.at[pl.ds(base + jj * 128 * _D, 128 * _D)], buf, sem
            )

        def compute(jj, buf, sem):
            pltpu.make_async_copy(
                rows_hbm.at[pl.ds(base + jj * 128 * _D, 128 * _D)], buf, sem
            ).wait()
            for blk in range(4):
                cb = blk * 2 * _L * _D
                e2 = zero
                un = zero
                vn = zero
                for d in range(_D):
                    uc = plsc.load_gather(buf, [cb + u_base + d])
                    vc = plsc.load_gather(buf, [cb + v_base + d])
                    df = uc - vc
                    e2 = e2 + df * df
                    un = un + uc * uc
                    vn = vn + vc * vc
                xblk = 1.0 + 2.0 * e2 / ((1.0 - un) * (1.0 - vn))
                x_v[pl.ds((jj * 4 + blk) * _L, _L)] = xblk

        fire(0, buf0, sem0)
        fire(1, buf1, sem1)

        @pl.loop(0, kc, step=2)
        def _(j):
            compute(j, buf0, sem0)

            @pl.when(j + 2 < kc)
            def _():
                fire(j + 2, buf0, sem0)

            compute(j + 1, buf1, sem1)

            @pl.when(j + 3 < kc)
            def _():
                fire(j + 3, buf1, sem1)

        pltpu.sync_copy(x_v, out_hbm.at[pl.ds(wid * pairs_per_w, pairs_per_w)])

    return sc_kernel(rows)


def _tc_tail_body(x_ref, lab_ref, out_ref):
    x = x_ref[...]
    dist = jnp.log(x + jnp.sqrt(x * x - 1.0))
    z = jnp.exp((dist - _R) / _T)
    lab = lab_ref[...]
    out_ref[...] = jnp.where(lab == 1, jnp.log(z + 1.0), jnp.log(1.0 + 1.0 / z))


def _tc_tail(x, labels):
    b = labels.shape[0]
    r = b // 128
    out = pl.pallas_call(
        _tc_tail_body,
        out_shape=jax.ShapeDtypeStruct((r, 128), jnp.float32),
    )(x.reshape(r, 128), labels.reshape(r, 128))
    return out.reshape(b)


@jax.jit
def kernel(pairs, labels, table):
    b = pairs.shape[0]
    tT = table.T
    idx = pairs.reshape(-1)
    rows = _sc_scatter_rows(tT, idx)
    x = _sc_pair_x(rows, b)
    return _tc_tail(x, labels)
